# Initial kernel scaffold; baseline (speedup 1.0000x reference)
import jax
import jax.numpy as jnp
from jax.experimental import pallas as pl


def kernel(prot_coord_x, prot_scalar_x, prot_pos, prot_edge_index, prot_edge_attr, prot_batch, mol_coord_x, mol_scalar_x, mol_pos, mol_edge_index, mol_edge_attr, mol_batch, params):
    raise NotImplementedError("write your pallas kernel here")


# speedup vs baseline: 1.0727x; 1.0727x over previous
import functools

import jax
import jax.numpy as jnp
import numpy as np
from jax.experimental import pallas as pl
from jax.experimental.pallas import tpu as pltpu

HID = 256
HEADS = 4
HEAD_DIM = HID // HEADS
NGRAPH = 32
PATHS = 3


def _ln(x, g, b):
    mu = x.mean(-1, keepdims=True)
    var = ((x - mu) ** 2).mean(-1, keepdims=True)
    return (x - mu) / jnp.sqrt(var + 1e-5) * g + b


def _seg_softmax_flat(logits, seg, n):
    m = jax.ops.segment_max(logits, seg, num_segments=n)
    m = jnp.where(jnp.isfinite(m), m, 0.0)
    e = jnp.exp(logits - m[seg])
    s = jax.ops.segment_sum(e, seg, num_segments=n)
    return e / (s[seg] + 1e-16)


def _edge_mlp_body(pre_ref, w2_ref, b2_ref, wx_ref, bx_ref, m2_ref, coef_ref):
    x = pre_ref[...]
    x = x * jax.nn.sigmoid(x)
    m2 = x @ w2_ref[...] + b2_ref[...]
    m2 = m2 * jax.nn.sigmoid(m2)
    m2_ref[...] = m2
    coef_ref[...] = m2 @ wx_ref[...] + bx_ref[...]


def _edge_mlp(pre_full, w2, b2, wx, bx, block_e=512):
    e_num = pre_full.shape[0]
    assert e_num % block_e == 0, e_num
    grid = (e_num // block_e,)
    m2, coef = pl.pallas_call(
        _edge_mlp_body,
        grid=grid,
        in_specs=[
            pl.BlockSpec((block_e, HID), lambda i: (i, 0)),
            pl.BlockSpec((HID, HID), lambda i: (0, 0)),
            pl.BlockSpec((1, HID), lambda i: (0, 0)),
            pl.BlockSpec((HID, 1), lambda i: (0, 0)),
            pl.BlockSpec((1, 1), lambda i: (0, 0)),
        ],
        out_specs=[
            pl.BlockSpec((block_e, HID), lambda i: (i, 0)),
            pl.BlockSpec((block_e, 1), lambda i: (i, 0)),
        ],
        out_shape=[
            jax.ShapeDtypeStruct((e_num, HID), jnp.float32),
            jax.ShapeDtypeStruct((e_num, 1), jnp.float32),
        ],
    )(pre_full, w2, b2.reshape(1, HID), wx, bx.reshape(1, 1))
    return m2, coef


def _gat_logits_body(xls_ref, xrd_ref, att_ref, sel_ref, out_ref):
    x = xls_ref[...] + xrd_ref[...]
    x = jnp.where(x >= 0.0, x, 0.2 * x)
    out_ref[...] = (x * att_ref[...]) @ sel_ref[...]


def _gat_logits(xls, xrd, att_row, sel, block_e=512):
    e_num, dcat = xls.shape
    nh = sel.shape[1]
    grid = (e_num // block_e,)
    return pl.pallas_call(
        _gat_logits_body,
        grid=grid,
        in_specs=[
            pl.BlockSpec((block_e, dcat), lambda i: (i, 0)),
            pl.BlockSpec((block_e, dcat), lambda i: (i, 0)),
            pl.BlockSpec((1, dcat), lambda i: (0, 0)),
            pl.BlockSpec((dcat, nh), lambda i: (0, 0)),
        ],
        out_specs=pl.BlockSpec((block_e, nh), lambda i: (i, 0)),
        out_shape=jax.ShapeDtypeStruct((e_num, nh), jnp.float32),
    )(xls, xrd, att_row, sel)


def _gat_weight_body(xls_ref, alpha_ref, selt_ref, out_ref):
    out_ref[...] = xls_ref[...] * (alpha_ref[...] @ selt_ref[...])


def _gat_weight(xls, alpha, selt, block_e=512):
    e_num, dcat = xls.shape
    nh = alpha.shape[1]
    grid = (e_num // block_e,)
    return pl.pallas_call(
        _gat_weight_body,
        grid=grid,
        in_specs=[
            pl.BlockSpec((block_e, dcat), lambda i: (i, 0)),
            pl.BlockSpec((block_e, nh), lambda i: (i, 0)),
            pl.BlockSpec((nh, dcat), lambda i: (0, 0)),
        ],
        out_specs=pl.BlockSpec((block_e, dcat), lambda i: (i, 0)),
        out_shape=jax.ShapeDtypeStruct((e_num, dcat), jnp.float32),
    )(xls, alpha, selt)


def _attn_body(pe_ref, k_ref, v_ref, pb_ref, mb_ref, wq_ref, bq_ref,
               wg1_ref, bg1_ref, wg2_ref, bg2_ref, fused_ref, gate_ref):
    pe = pe_ref[...]
    q = pe @ wq_ref[...] + bq_ref[...]
    s = jax.lax.dot_general(q, k_ref[...], (((1,), (1,)), ((), ())))
    s = s * (1.0 / np.sqrt(float(HID)))
    mask = pb_ref[...] == mb_ref[...]
    s = jnp.where(mask, s, -1e9)
    m = jnp.max(s, axis=1, keepdims=True)
    e = jnp.exp(s - m)
    denom = jnp.sum(e, axis=1, keepdims=True)
    attn = e / denom
    fused = pe + attn @ v_ref[...]
    fused_ref[...] = fused
    g1 = fused @ wg1_ref[...] + bg1_ref[...]
    g1 = jnp.maximum(g1, 0.0)
    gate_ref[...] = g1 @ wg2_ref[...] + bg2_ref[...]


def _attn_gate(pe, kmat, vmat, pb, mb, wq, bq, wg1, bg1, wg2, bg2,
               block_p=512):
    n_prot = pe.shape[0]
    n_mol = kmat.shape[0]
    n_pad = ((n_prot + block_p - 1) // block_p) * block_p
    pe_p = jnp.pad(pe, ((0, n_pad - n_prot), (0, 0)))
    pb_p = jnp.pad(pb, (0, n_pad - n_prot), constant_values=-1)
    grid = (n_pad // block_p,)
    fused, gate = pl.pallas_call(
        _attn_body,
        grid=grid,
        in_specs=[
            pl.BlockSpec((block_p, HID), lambda i: (i, 0)),
            pl.BlockSpec((n_mol, HID), lambda i: (0, 0)),
            pl.BlockSpec((n_mol, HID), lambda i: (0, 0)),
            pl.BlockSpec((block_p, 1), lambda i: (i, 0)),
            pl.BlockSpec((1, n_mol), lambda i: (0, 0)),
            pl.BlockSpec((HID, HID), lambda i: (0, 0)),
            pl.BlockSpec((1, HID), lambda i: (0, 0)),
            pl.BlockSpec((HID, HID // 2), lambda i: (0, 0)),
            pl.BlockSpec((1, HID // 2), lambda i: (0, 0)),
            pl.BlockSpec((HID // 2, 1), lambda i: (0, 0)),
            pl.BlockSpec((1, 1), lambda i: (0, 0)),
        ],
        out_specs=[
            pl.BlockSpec((block_p, HID), lambda i: (i, 0)),
            pl.BlockSpec((block_p, 1), lambda i: (i, 0)),
        ],
        out_shape=[
            jax.ShapeDtypeStruct((n_pad, HID), jnp.float32),
            jax.ShapeDtypeStruct((n_pad, 1), jnp.float32),
        ],
    )(pe_p, kmat, vmat, pb_p.reshape(-1, 1), mb.reshape(1, -1),
      wq, bq.reshape(1, HID), wg1, bg1.reshape(1, HID // 2),
      wg2, bg2.reshape(1, 1))
    return fused[:n_prot], gate[:n_prot, 0]


def _egnn_layer(lp, h, pos, src, dst, edge_attr, n, din):
    w1 = lp["phi_e1"]["w"]
    b1 = lp["phi_e1"]["b"]
    hd = h @ w1[:din]
    hs = h @ w1[din:2 * din]
    w_d2 = w1[2 * din]
    w_ea = w1[2 * din + 1:]
    rel = pos[dst] - pos[src]
    d2 = (rel ** 2).sum(-1, keepdims=True)
    pre = hd[dst] + hs[src] + d2 * w_d2[None, :] + edge_attr @ w_ea + b1[None, :]
    m2, coef = _edge_mlp(pre, lp["phi_e2"]["w"], lp["phi_e2"]["b"],
                         lp["phi_x"]["w"], lp["phi_x"]["b"])
    pos_new = pos + jax.ops.segment_sum(rel * coef, dst, num_segments=n) / 32.0
    agg = jax.ops.segment_sum(m2, dst, num_segments=n)
    wh = lp["phi_h"]["w"]
    h_new = h @ wh[:din] + agg @ wh[din:] + lp["phi_h"]["b"][None, :]
    h_new = _ln(h_new, lp["ln_g"], lp["ln_b"])
    return h_new, pos_new


def _gat_block(bp, s, src, dst, n, sel, selt):
    nh = PATHS * HEADS
    wl = jnp.concatenate([p["wl"] for p in bp["paths"]], axis=1)
    wr = jnp.concatenate([p["wr"] for p in bp["paths"]], axis=1)
    att_row = jnp.concatenate(
        [p["att"].reshape(1, HID) for p in bp["paths"]], axis=1)
    xl = s @ wl
    xr = s @ wr
    xls = xl[src]
    xrd = xr[dst]
    logits = _gat_logits(xls, xrd, att_row, sel)
    e_num = logits.shape[0]
    seg_flat = dst[:, None] * nh + jnp.arange(nh, dtype=dst.dtype)[None, :]
    alpha = _seg_softmax_flat(logits.reshape(-1), seg_flat.reshape(-1),
                              n * nh).reshape(e_num, nh)
    weighted = _gat_weight(xls, alpha, selt)
    osum = jax.ops.segment_sum(weighted, dst, num_segments=n)
    omean = (osum[:, :HID] + osum[:, HID:2 * HID] + osum[:, 2 * HID:]) \
        / float(PATHS)
    return _ln(jnp.maximum(omean, 0.0), bp["ln_g"], bp["ln_b"])


def _encode(p, coord_x, scalar_x, pos, edge_index, edge_attr, sel, selt):
    src, dst = edge_index[0], edge_index[1]
    n = coord_x.shape[0]
    h, pp = coord_x, pos
    dims = [coord_x.shape[1], HID]
    for lp, din in zip(p["egnn"], dims):
        h, pp = _egnn_layer(lp, h, pp, src, dst, edge_attr, n, din)
    s = scalar_x
    for bp in p["gat"]:
        s = _gat_block(bp, s, src, dst, n, sel, selt)
    fw = p["fusion"]["w"]
    fused = h @ fw[:HID] + s @ fw[HID:] + p["fusion"]["b"][None, :]
    return jax.nn.relu(_ln(fused, p["f_ln_g"], p["f_ln_b"]))


def kernel(prot_coord_x, prot_scalar_x, prot_pos, prot_edge_index,
           prot_edge_attr, prot_batch, mol_coord_x, mol_scalar_x, mol_pos,
           mol_edge_index, mol_edge_attr, mol_batch, params):
    nh = PATHS * HEADS
    sel = (jnp.arange(PATHS * HID)[:, None] // HEAD_DIM
           == jnp.arange(nh)[None, :]).astype(jnp.float32)
    selt = sel.T
    pe = _encode(params["prot"], prot_coord_x, prot_scalar_x, prot_pos,
                 prot_edge_index, prot_edge_attr, sel, selt)
    me = _encode(params["mol"], mol_coord_x, mol_scalar_x, mol_pos,
                 mol_edge_index, mol_edge_attr, sel, selt)
    kmat = me @ params["wk"]["w"] + params["wk"]["b"]
    vmat = me @ params["wv"]["w"] + params["wv"]["b"]
    fused, gate = _attn_gate(
        pe, kmat, vmat, prot_batch, mol_batch,
        params["wq"]["w"], params["wq"]["b"],
        params["gate1"]["w"], params["gate1"]["b"],
        params["gate2"]["w"], params["gate2"]["b"])
    gw = _seg_softmax_flat(gate, prot_batch, NGRAPH)
    pooled = jax.ops.segment_sum(fused * gw[:, None], prot_batch,
                                 num_segments=NGRAPH)
    o1 = jnp.maximum(pooled @ params["out1"]["w"] + params["out1"]["b"], 0.0)
    return o1 @ params["out2"]["w"] + params["out2"]["b"]
